# Initial kernel scaffold; baseline (speedup 1.0000x reference)
import jax
import jax.numpy as jnp
from jax.experimental import pallas as pl


def kernel(ref_pt_list, slots, Wq, bq, Wk, bk, Wv, bv, Wo, bo):
    raise NotImplementedError("write your pallas kernel here")


# speedup vs baseline: 10.0039x; 10.0039x over previous
import functools

import jax
import jax.numpy as jnp
from jax import lax
from jax.experimental import pallas as pl
from jax.experimental.pallas import tpu as pltpu
from jax.experimental.pallas import tpu_sc as plsc

B, NS, NR, C, K, ITERS = 16, 256, 4096, 256, 16, 3

_F32 = jnp.float32
_BF16 = jnp.bfloat16


def _split3(x):
    x1 = x.astype(_BF16)
    r1 = x - x1.astype(_F32)
    x2 = r1.astype(_BF16)
    x3 = (r1 - x2.astype(_F32)).astype(_BF16)
    return x1, x2, x3


def _dot1(a, b, dims):
    return lax.dot_general(
        a.astype(_BF16), b.astype(_BF16), dims, preferred_element_type=_F32
    )


def _topk_body(s_ref, nr_ref, idx_ref):
    b = pl.program_id(0)
    s = s_ref[0]
    nr = nr_ref[0]
    corr = lax.dot_general(
        s.astype(_BF16), nr.astype(_BF16), (((1,), (1,)), ((), ())),
        preferred_element_type=_F32,
    )
    col = lax.broadcasted_iota(jnp.int32, (NS, NR), 1)
    neg = _F32(-jnp.inf)
    for t in range(K):
        m = jnp.max(corr, axis=1, keepdims=True)
        am = jnp.min(jnp.where(corr == m, col, NR), axis=1, keepdims=True)
        idx_ref[0, :, t : t + 1] = am + b * NR
        corr = jnp.where(col == am, neg, corr)


_topk = pl.pallas_call(
    _topk_body,
    grid=(B,),
    in_specs=[
        pl.BlockSpec((1, NS, C), lambda b: (b, 0, 0)),
        pl.BlockSpec((1, NR, C), lambda b: (b, 0, 0)),
    ],
    out_specs=pl.BlockSpec((1, NS, K), lambda b: (b, 0, 0)),
    out_shape=jax.ShapeDtypeStruct((B, NS, K), jnp.int32),
)


_NW = 32
_ROWS = B * NS * K
_RPW = _ROWS // _NW
_CH = 128
_NCH = _RPW // _CH


def _gather_body(table_hbm, idx_hbm, out_hbm, idx_v, buf0, buf1, sem0, sem1):
    wid = lax.axis_index("s") * 2 + lax.axis_index("c")
    cbase = wid * _NCH
    pltpu.sync_copy(idx_hbm.at[pl.ds(cbase, _NCH)], idx_v)
    bufs = (buf0, buf1)
    sems = (sem0, sem1)
    gh = [None] * _NCH
    wh = [None] * _NCH
    gh[0] = pltpu.async_copy(table_hbm.at[idx_v.at[0]], bufs[0], sems[0])
    if _NCH > 1:
        gh[1] = pltpu.async_copy(table_hbm.at[idx_v.at[1]], bufs[1], sems[1])
    for c in range(_NCH):
        p = c & 1
        gh[c].wait()
        wh[c] = pltpu.async_copy(
            bufs[p], out_hbm.at[pl.ds((cbase + c) * _CH, _CH)], sems[p]
        )
        if c + 2 < _NCH:
            wh[c].wait()
            gh[c + 2] = pltpu.async_copy(
                table_hbm.at[idx_v.at[c + 2]], bufs[p], sems[p]
            )
    for c in range(max(_NCH - 2, 0), _NCH):
        if wh[c] is not None:
            wh[c].wait()


@functools.cache
def _make_gather():
    return functools.partial(
        pl.kernel,
        mesh=plsc.VectorSubcoreMesh(core_axis_name="c", subcore_axis_name="s"),
        out_type=jax.ShapeDtypeStruct((_ROWS, C), _F32),
        scratch_types=[
            pltpu.VMEM((_NCH, _CH), jnp.int32),
            pltpu.VMEM((_CH, C), _F32),
            pltpu.VMEM((_CH, C), _F32),
            pltpu.SemaphoreType.DMA,
            pltpu.SemaphoreType.DMA,
        ],
    )(_gather_body)


def _attn_body(s_ref, gath_ref, wq_ref, bq_ref, wk_ref, bk_ref, wv_ref, bv_ref,
               wo_ref, bo_ref, out_ref):
    nt = (((1,), (1,)), ((), ()))
    s = s_ref[0]
    q = _dot1(s, wq_ref[...], nt) + bq_ref[...]
    g2 = gath_ref[0]
    kk = _dot1(g2, wk_ref[...], nt) + bk_ref[...]
    vv = _dot1(g2, wv_ref[...], nt) + bv_ref[...]
    q1, q2, q3 = _split3(q)
    k1, k2, k3 = _split3(kk)
    qcat = jnp.concatenate([q1, q1, q2, q1, q2, q3], axis=1)
    kcat = jnp.concatenate([k1, k2, k1, k3, k2, k1], axis=1)
    la = lax.dot_general(qcat, kcat, nt, preferred_element_type=_F32)
    row = lax.broadcasted_iota(jnp.int32, (NS, NS * K), 0)
    colm = lax.broadcasted_iota(jnp.int32, (NS, NS * K), 1)
    cols = []
    for j in range(K):
        sel = jnp.where(colm == row * K + j, la, _F32(0.0))
        cols.append(jnp.sum(sel, axis=1, keepdims=True))
    logits = jnp.concatenate(cols, axis=1) / _F32(C ** 0.5)
    mx = jnp.max(logits, axis=1, keepdims=True)
    e = jnp.exp(logits - mx)
    attn = e / jnp.sum(e, axis=1, keepdims=True)
    vv3 = vv.reshape(NS, K, C)
    agg = vv3[:, 0, :] * attn[:, 0:1]
    for j in range(1, K):
        agg = agg + vv3[:, j, :] * attn[:, j : j + 1]
    out_ref[0] = s + _dot1(agg, wo_ref[...], nt) + bo_ref[...]


_attn = pl.pallas_call(
    _attn_body,
    grid=(B,),
    in_specs=[
        pl.BlockSpec((1, NS, C), lambda b: (b, 0, 0)),
        pl.BlockSpec((1, NS * K, C), lambda b: (b, 0, 0)),
        pl.BlockSpec((C, C), lambda b: (0, 0)),
        pl.BlockSpec((1, C), lambda b: (0, 0)),
        pl.BlockSpec((C, C), lambda b: (0, 0)),
        pl.BlockSpec((1, C), lambda b: (0, 0)),
        pl.BlockSpec((C, C), lambda b: (0, 0)),
        pl.BlockSpec((1, C), lambda b: (0, 0)),
        pl.BlockSpec((C, C), lambda b: (0, 0)),
        pl.BlockSpec((1, C), lambda b: (0, 0)),
    ],
    out_specs=pl.BlockSpec((1, NS, C), lambda b: (b, 0, 0)),
    out_shape=jax.ShapeDtypeStruct((B, NS, C), _F32),
)


def kernel(ref_pt_list, slots, Wq, bq, Wk, bk, Wv, bv, Wo, bo):
    ref = ref_pt_list[-1]
    nr = ref / jnp.linalg.norm(ref, axis=2, keepdims=True)
    table = ref.reshape(B * NR, C)
    b_ = lambda v: v.reshape(1, C)
    s = slots
    for it in range(ITERS):
        s_n = s / jnp.linalg.norm(s, axis=2, keepdims=True)
        idx = _topk(s_n, nr)
        gath = _make_gather()(table, idx.reshape(_ROWS // _CH, _CH))
        if it < ITERS - 1:
            ref_f = jnp.transpose(gath.reshape(B, NS, K, C), (0, 3, 1, 2))
            s_c = jnp.transpose(s, (0, 2, 1))
            q = jnp.einsum('dc,bcn->bdn', Wq, s_c) + bq[None, :, None]
            kk = jnp.einsum('dc,bcnk->bdnk', Wk, ref_f) + bk[None, :, None, None]
            vv = jnp.einsum('dc,bcnk->bdnk', Wv, ref_f) + bv[None, :, None, None]
            lg = jnp.einsum('bcn,bcnk->bnk', q, kk) / jnp.sqrt(_F32(C))
            at = jax.nn.softmax(lg, axis=-1)
            ag = jnp.einsum('bnk,bcnk->bcn', at, vv)
            s_c = s_c + jnp.einsum('dc,bcn->bdn', Wo, ag) + bo[None, :, None]
            s = jnp.transpose(s_c, (0, 2, 1))
        else:
            s = _attn(s, gath.reshape(B, NS * K, C), Wq, b_(bq), Wk, b_(bk),
                      Wv, b_(bv), Wo, b_(bo))
    return s
